# baseline (speedup 1.0000x reference)
import jax
import jax.numpy as jnp
from jax.experimental import pallas as pl
from jax.experimental.pallas import tpu as pltpu

BATCH = 16
CHUNKS = 8

def _body(emb_any, out_any, scratch, load_sems, sems):
    f = scratch.shape[0]
    h = f // CHUNKS
    for s in range(CHUNKS):
        pltpu.make_async_copy(
            emb_any.at[pl.ds(s * h, h)], scratch.at[pl.ds(s * h, h)], load_sems.at[s]
        ).start()
    for s in range(CHUNKS):
        pltpu.make_async_copy(
            emb_any.at[pl.ds(s * h, h)], scratch.at[pl.ds(s * h, h)], load_sems.at[s]
        ).wait()
        for b in range(BATCH):
            pltpu.make_async_copy(
                scratch.at[pl.ds(s * h, h)],
                out_any.at[b, pl.ds(s * h, h)],
                sems.at[b, s],
            ).start()
    for b in range(BATCH):
        for s in range(CHUNKS):
            pltpu.make_async_copy(
                scratch.at[pl.ds(s * h, h)],
                out_any.at[b, pl.ds(s * h, h)],
                sems.at[b, s],
            ).wait()

def kernel(x, grid_embedding):
    batch = x.shape[0]
    g2, f = grid_embedding.shape
    emb_t = grid_embedding.T
    out_t = pl.pallas_call(
        _body,
        in_specs=[pl.BlockSpec(memory_space=pl.ANY)],
        out_specs=pl.BlockSpec(memory_space=pl.ANY),
        out_shape=jax.ShapeDtypeStruct((batch, f, g2), grid_embedding.dtype),
        scratch_shapes=[
            pltpu.VMEM((f, g2), grid_embedding.dtype),
            pltpu.SemaphoreType.DMA((CHUNKS,)),
            pltpu.SemaphoreType.DMA((BATCH, CHUNKS)),
        ],
    )(emb_t)
    return jnp.transpose(out_t, (0, 2, 1))


# speedup vs baseline: 1.0391x; 1.0391x over previous
import jax
import jax.numpy as jnp
from jax.experimental import pallas as pl
from jax.experimental.pallas import tpu as pltpu

BATCH = 16
CHUNKS = 2

def _body(emb_any, out_any, scratch, load_sems, sems):
    f = scratch.shape[0]
    h = f // CHUNKS
    for s in range(CHUNKS):
        pltpu.make_async_copy(
            emb_any.at[pl.ds(s * h, h)], scratch.at[pl.ds(s * h, h)], load_sems.at[s]
        ).start()
    for s in range(CHUNKS):
        pltpu.make_async_copy(
            emb_any.at[pl.ds(s * h, h)], scratch.at[pl.ds(s * h, h)], load_sems.at[s]
        ).wait()
        for b in range(BATCH):
            pltpu.make_async_copy(
                scratch.at[pl.ds(s * h, h)],
                out_any.at[b, pl.ds(s * h, h)],
                sems.at[b, s],
            ).start()
    for b in range(BATCH):
        for s in range(CHUNKS):
            pltpu.make_async_copy(
                scratch.at[pl.ds(s * h, h)],
                out_any.at[b, pl.ds(s * h, h)],
                sems.at[b, s],
            ).wait()

def kernel(x, grid_embedding):
    batch = x.shape[0]
    g2, f = grid_embedding.shape
    emb_t = grid_embedding.T
    out_t = pl.pallas_call(
        _body,
        in_specs=[pl.BlockSpec(memory_space=pl.ANY)],
        out_specs=pl.BlockSpec(memory_space=pl.ANY),
        out_shape=jax.ShapeDtypeStruct((batch, f, g2), grid_embedding.dtype),
        scratch_shapes=[
            pltpu.VMEM((f, g2), grid_embedding.dtype),
            pltpu.SemaphoreType.DMA((CHUNKS,)),
            pltpu.SemaphoreType.DMA((BATCH, CHUNKS)),
        ],
    )(emb_t)
    return jnp.transpose(out_t, (0, 2, 1))
